# baseline (speedup 1.0000x reference)
import functools

import jax
import jax.numpy as jnp
from jax import lax
from jax.experimental import pallas as pl
from jax.experimental.pallas import tpu as pltpu
from jax.experimental.pallas import tpu_sc as plsc

_ENT = 1000000
_REL = 1000
_DIM = 64
_BATCH = 16384
_NC = 2
_NS = 16
_NW = _NC * _NS
_BPW = _BATCH // _NW
_CH = 128
_NCH = _BPW // _CH


def _rsqrt16(s):
    i = plsc.bitcast(s, jnp.int32)
    i = jnp.int32(0x5F3759DF) - lax.shift_right_logical(i, 1)
    y = plsc.bitcast(i, jnp.float32)
    half = s * 0.5
    for _ in range(3):
        y = y * (1.5 - half * y * y)
    return y


def kernel(ent_embeddings, rel_embeddings, batch_h, batch_t, batch_r):
    mesh = plsc.VectorSubcoreMesh(core_axis_name="c", subcore_axis_name="s")

    @functools.partial(
        pl.kernel,
        out_type=jax.ShapeDtypeStruct((_BATCH,), jnp.float32),
        mesh=mesh,
        compiler_params=pltpu.CompilerParams(
            needs_layout_passes=False, use_tc_tiling_on_sc=False),
        scratch_types=[
            pltpu.VMEM((_BPW,), jnp.int32),
            pltpu.VMEM((_BPW,), jnp.int32),
            pltpu.VMEM((_BPW,), jnp.int32),
            pltpu.VMEM((_BPW, _DIM), jnp.float32),
            pltpu.VMEM((_BPW, _DIM), jnp.float32),
            pltpu.VMEM((_BPW, _DIM), jnp.float32),
            pltpu.VMEM((_BPW,), jnp.float32),
            pltpu.SemaphoreType.DMA,
        ],
    )
    def k(ent_hbm, rel_hbm, bh_hbm, bt_hbm, br_hbm, out_hbm,
          ih_v, it_v, ir_v, h_v, t_v, r_v, o_v, sem):
        wid = lax.axis_index("s") * _NC + lax.axis_index("c")
        base = wid * _BPW

        pltpu.sync_copy(bh_hbm.at[pl.ds(base, _BPW)], ih_v)
        pltpu.sync_copy(bt_hbm.at[pl.ds(base, _BPW)], it_v)
        pltpu.sync_copy(br_hbm.at[pl.ds(base, _BPW)], ir_v)

        copies = []
        for j in range(_NCH):
            sl = pl.ds(j * _CH, _CH)
            copies.append(pltpu.async_copy(
                ent_hbm.at[ih_v.at[sl]], h_v.at[sl], sem))
            copies.append(pltpu.async_copy(
                ent_hbm.at[it_v.at[sl]], t_v.at[sl], sem))
            copies.append(pltpu.async_copy(
                rel_hbm.at[ir_v.at[sl]], r_v.at[sl], sem))
        for c in copies:
            c.wait()

        def row(i, _):
            sh = jnp.zeros((16,), jnp.float32)
            st = jnp.zeros((16,), jnp.float32)
            sr = jnp.zeros((16,), jnp.float32)
            hs, ts, rs = [], [], []
            for kk in range(_DIM // 16):
                sl = pl.ds(kk * 16, 16)
                hv = h_v[i, sl]
                tv = t_v[i, sl]
                rv = r_v[i, sl]
                hs.append(hv)
                ts.append(tv)
                rs.append(rv)
                sh = sh + hv * hv
                st = st + tv * tv
                sr = sr + rv * rv
            eps = jnp.float32(1e-24)
            inv_h = _rsqrt16(jnp.full((16,), jnp.maximum(jnp.sum(sh), eps)))
            inv_t = _rsqrt16(jnp.full((16,), jnp.maximum(jnp.sum(st), eps)))
            inv_r = _rsqrt16(jnp.full((16,), jnp.maximum(jnp.sum(sr), eps)))
            acc = jnp.zeros((16,), jnp.float32)
            for kk in range(_DIM // 16):
                acc = acc + jnp.abs(hs[kk] * inv_h + rs[kk] * inv_r
                                    - ts[kk] * inv_t)
            lane = lax.iota(jnp.int32, 16)
            plsc.store_scatter(o_v, [jnp.full((16,), i, jnp.int32)],
                               plsc.cumsum(acc), mask=lane == 15)
            return 0

        lax.fori_loop(0, _BPW, row, 0)

        pltpu.sync_copy(o_v, out_hbm.at[pl.ds(base, _BPW)])

    return k(ent_embeddings, rel_embeddings, batch_h, batch_t, batch_r)
